# baseline (speedup 1.0000x reference)
import functools

import jax
import jax.numpy as jnp
from jax import lax
from jax.experimental import pallas as pl
from jax.experimental.pallas import tpu as pltpu
from jax.experimental.pallas import tpu_sc as plsc

NN = 10000
NE = 5000
NI = 320000
D = 128
DH = 64

NC = 2
NS = 16
CH = 32

NNP = 10112
NEP = 5120
NIP = 323584
NCH16 = NIP // (NS * CH)
CHD = 128
NCHD = NIP // (NS * CHD)
DEGW = 8
NBUF = 4

_mesh = functools.partial(
    plsc.VectorSubcoreMesh, core_axis_name="c", subcore_axis_name="s",
    num_cores=NC, num_subcores=NS)
_sc_params = pltpu.CompilerParams(use_tc_tiling_on_sc=False)


def _zero_rows(buf, nrows, width):
    def row(i, carry):
        for k in range(width // 16):
            buf[i, pl.ds(k * 16, 16)] = jnp.zeros((16,), jnp.float32)
        return carry
    lax.fori_loop(0, nrows, row, 0)


def _degree_kernel(nd_hbm, he_hbm, ones_hbm, zeros_hbm, degv_out, dege_out,
                   degv_acc, dege_acc, idx_v, ones_v, ssem):
    c = lax.axis_index("c")
    s = lax.axis_index("s")

    pltpu.sync_copy(ones_hbm, ones_v)

    @pl.when(c == 0)
    def _():
        pltpu.sync_copy(zeros_hbm, degv_acc.at[pl.ds(s * 632, 632)])
        pltpu.sync_copy(nd_hbm.at[s], idx_v)

    @pl.when(c == 1)
    def _():
        pltpu.sync_copy(zeros_hbm.at[pl.ds(0, 320)],
                        dege_acc.at[pl.ds(s * 320, 320)])
        pltpu.sync_copy(he_hbm.at[s], idx_v)

    plsc.subcore_barrier()

    def scatter_all(acc):
        def chunk(j, carry):
            p = lax.rem(j, 4)

            @pl.when(j >= 4)
            def _():
                pltpu.make_async_copy(
                    ones_v, acc.at[idx_v.at[j]], ssem.at[p]).wait()

            pltpu.async_copy(ones_v, acc.at[idx_v.at[j]], ssem.at[p],
                             add=True)
            return carry
        lax.fori_loop(0, NCHD, chunk, 0)
        for k in range(4):
            pltpu.make_async_copy(
                ones_v, acc.at[idx_v.at[NCHD - 1]], ssem.at[k]).wait()

    @pl.when(c == 0)
    def _():
        scatter_all(degv_acc)

    @pl.when(c == 1)
    def _():
        scatter_all(dege_acc)

    plsc.subcore_barrier()

    @pl.when(c == 0)
    def _():
        pltpu.sync_copy(degv_acc.at[pl.ds(s * 632, 632)],
                        degv_out.at[pl.ds(s * 632, 632)])

    @pl.when(c == 1)
    def _():
        pltpu.sync_copy(dege_acc.at[pl.ds(s * 320, 320)],
                        dege_out.at[pl.ds(s * 320, 320)])


def _degrees(nd3, he3):
    return pl.kernel(
        _degree_kernel,
        out_type=[jax.ShapeDtypeStruct((NNP, DEGW), jnp.float32),
                  jax.ShapeDtypeStruct((NEP, DEGW), jnp.float32)],
        mesh=_mesh(),
        compiler_params=_sc_params,
        scratch_types=[
            pltpu.VMEM_SHARED((NNP, DEGW), jnp.float32),
            pltpu.VMEM_SHARED((NEP, DEGW), jnp.float32),
            pltpu.VMEM((NCHD, CHD), jnp.int32),
            pltpu.VMEM((CHD, DEGW), jnp.float32),
            pltpu.SemaphoreType.DMA((4,)),
        ],
    )(nd3, he3, jnp.ones((CHD, DEGW), jnp.float32),
      jnp.zeros((632, DEGW), jnp.float32))


def _mul_rows(dst, src, nrows):
    def row(i, carry):
        for k in range(DH // 16):
            sl = pl.ds(k * 16, 16)
            dst[i, sl] = dst[i, sl] * src[i, sl]
        return carry
    lax.fori_loop(0, nrows, row, 0)


def _make_spmm(n_seg, rows_per_tile, n_tbl, tbl_rows_per_tile, nbuf=NBUF,
               norm=False):
    def body(*refs):
        if norm:
            (t0_hbm, t1_hbm, dexp_hbm, gidx_hbm, sidx_hbm, o0, o1,
             acc, tbl, g_v, s_v, rowbuf, gsem, ssem) = refs
        else:
            (t0_hbm, t1_hbm, gidx_hbm, sidx_hbm, o0, o1,
             acc, tbl, g_v, s_v, rowbuf, gsem, ssem) = refs
        c = lax.axis_index("c")
        s = lax.axis_index("s")

        _zero_rows(rowbuf.at[0], CH, DH)
        base = s * rows_per_tile
        off = 0
        while off < rows_per_tile:
            n = min(CH, rows_per_tile - off)
            pltpu.sync_copy(rowbuf.at[0].at[pl.ds(0, n)],
                            acc.at[pl.ds(base + off, n)])
            off += n

        if not norm:
            tsl = pl.ds(s * tbl_rows_per_tile, tbl_rows_per_tile)

            @pl.when(c == 0)
            def _():
                pltpu.sync_copy(t0_hbm.at[tsl], tbl.at[tsl])

            @pl.when(c == 1)
            def _():
                pltpu.sync_copy(t1_hbm.at[tsl], tbl.at[tsl])
        else:
            off = 0
            while off < tbl_rows_per_tile:
                n = min(CH, tbl_rows_per_tile - off)
                rsl = pl.ds(s * tbl_rows_per_tile + off, n)
                bsl = pl.ds(0, n)

                @pl.when(c == 0)
                def _():
                    pltpu.sync_copy(t0_hbm.at[rsl], rowbuf.at[0].at[bsl])

                @pl.when(c == 1)
                def _():
                    pltpu.sync_copy(t1_hbm.at[rsl], rowbuf.at[0].at[bsl])

                pltpu.sync_copy(dexp_hbm.at[rsl], rowbuf.at[1].at[bsl])
                _mul_rows(rowbuf.at[0], rowbuf.at[1], n)
                pltpu.sync_copy(rowbuf.at[0].at[bsl], tbl.at[rsl])
                off += n

        pltpu.sync_copy(gidx_hbm.at[s], g_v)
        pltpu.sync_copy(sidx_hbm.at[s], s_v)
        plsc.subcore_barrier()

        def do_phase(t_ref):
            for g in range(nbuf - 1):
                pltpu.async_copy(t_ref.at[g_v.at[g]], rowbuf.at[g], gsem.at[g])

            def chunk(j, carry):
                p = lax.rem(j, nbuf)
                pq = lax.rem(j + nbuf - 1, nbuf)

                @pl.when(j >= 1)
                def _():
                    pltpu.make_async_copy(
                        rowbuf.at[pq], acc.at[s_v.at[j]], ssem.at[pq]).wait()

                @pl.when(j + nbuf - 1 < NCH16)
                def _():
                    pltpu.async_copy(t_ref.at[g_v.at[j + nbuf - 1]],
                                     rowbuf.at[pq], gsem.at[pq])

                pltpu.make_async_copy(
                    t_ref.at[g_v.at[j]], rowbuf.at[p], gsem.at[p]).wait()
                pltpu.async_copy(
                    rowbuf.at[p], acc.at[s_v.at[j]], ssem.at[p], add=True)
                return carry
            lax.fori_loop(0, NCH16, chunk, 0)
            lastp = (NCH16 - 1) % nbuf
            pltpu.make_async_copy(
                rowbuf.at[lastp], acc.at[s_v.at[NCH16 - 1]],
                ssem.at[lastp]).wait()

        do_phase(tbl)

        plsc.subcore_barrier()

        @pl.when(c == 0)
        def _():
            pltpu.sync_copy(acc.at[pl.ds(s * rows_per_tile, rows_per_tile)],
                            o0.at[pl.ds(s * rows_per_tile, rows_per_tile)])

        @pl.when(c == 1)
        def _():
            pltpu.sync_copy(acc.at[pl.ds(s * rows_per_tile, rows_per_tile)],
                            o1.at[pl.ds(s * rows_per_tile, rows_per_tile)])

    return pl.kernel(
        body,
        out_type=[jax.ShapeDtypeStruct((n_seg, DH), jnp.float32),
                  jax.ShapeDtypeStruct((n_seg, DH), jnp.float32)],
        mesh=_mesh(),
        compiler_params=_sc_params,
        scratch_types=[
            pltpu.VMEM_SHARED((n_seg, DH), jnp.float32),
            pltpu.VMEM_SHARED((n_tbl, DH), jnp.float32),
            pltpu.VMEM((NCH16, CH), jnp.int32),
            pltpu.VMEM((NCH16, CH), jnp.int32),
            pltpu.VMEM((nbuf, CH, DH), jnp.float32),
            pltpu.SemaphoreType.DMA((nbuf,)),
            pltpu.SemaphoreType.DMA((nbuf,)),
        ],
    )


def _xnorm_tc(x_ref, dv_ref, de_ref, o0, o1, dexp):
    d = dv_ref[pl.ds(0, NN), 0:1]
    inv = jnp.where(d > 0, 1.0 / d, 0.0)
    zpad = jnp.zeros((NNP - NN, DH), jnp.float32)
    o0[pl.ds(0, NN), :] = x_ref[:, :DH] * inv
    o0[pl.ds(NN, NNP - NN), :] = zpad
    o1[pl.ds(0, NN), :] = x_ref[:, DH:] * inv
    o1[pl.ds(NN, NNP - NN), :] = zpad
    de = de_ref[:, 0:1]
    inve = jnp.where(de > 0, 1.0 / de, 0.0)
    dexp[...] = jnp.broadcast_to(inve, (NEP, DH))


def kernel(X, Y, node_idx, hyperedge_idx):
    del Y
    pad = NIP - NI
    nd_flat = jnp.concatenate([node_idx, jnp.full((pad,), NN, jnp.int32)])
    he_flat = jnp.concatenate([hyperedge_idx, jnp.full((pad,), NE, jnp.int32)])
    nd = nd_flat.reshape(NS, NCH16, CH)
    he = he_flat.reshape(NS, NCH16, CH)
    degv, dege = _degrees(nd_flat.reshape(NS, NCHD, CHD),
                          he_flat.reshape(NS, NCHD, CHD))

    xn0, xn1, dexp = pl.pallas_call(
        _xnorm_tc,
        out_shape=[jax.ShapeDtypeStruct((NNP, DH), jnp.float32),
                   jax.ShapeDtypeStruct((NNP, DH), jnp.float32),
                   jax.ShapeDtypeStruct((NEP, DH), jnp.float32)],
    )(X, degv, dege)

    e0, e1 = _make_spmm(NEP, 320, NNP, 632, nbuf=12)(xn0, xn1, nd, he)

    n0, n1 = _make_spmm(NNP, 632, NEP, 320, nbuf=12, norm=True)(
        e0, e1, dexp, he, nd)

    node_feat = jnp.concatenate([n0[:NN], n1[:NN]], axis=1)
    edge_feat = jnp.concatenate([e0[:NE], e1[:NE]], axis=1)
    return node_feat, edge_feat


# speedup vs baseline: 1.1025x; 1.1025x over previous
import functools

import jax
import jax.numpy as jnp
from jax import lax
from jax.experimental import pallas as pl
from jax.experimental.pallas import tpu as pltpu
from jax.experimental.pallas import tpu_sc as plsc

NN = 10000
NE = 5000
NI = 320000
D = 128
DH = 64

NC = 2
NS = 16
CH = 64

NNP = 10112
NEP = 5120
NIP = 323584
NCH16 = NIP // (NS * CH)
CHD = 128
NCHD = NIP // (NS * CHD)
DEGW = 8
NBUF = 4

_mesh = functools.partial(
    plsc.VectorSubcoreMesh, core_axis_name="c", subcore_axis_name="s",
    num_cores=NC, num_subcores=NS)
_sc_params = pltpu.CompilerParams(use_tc_tiling_on_sc=False)


def _zero_rows(buf, nrows, width):
    def row(i, carry):
        for k in range(width // 16):
            buf[i, pl.ds(k * 16, 16)] = jnp.zeros((16,), jnp.float32)
        return carry
    lax.fori_loop(0, nrows, row, 0)


def _degree_kernel(nd_hbm, he_hbm, ones_hbm, zeros_hbm, degv_out, dege_out,
                   degv_acc, dege_acc, idx_v, ones_v, ssem):
    c = lax.axis_index("c")
    s = lax.axis_index("s")

    pltpu.sync_copy(ones_hbm, ones_v)

    @pl.when(c == 0)
    def _():
        pltpu.sync_copy(zeros_hbm, degv_acc.at[pl.ds(s * 632, 632)])
        pltpu.sync_copy(nd_hbm.at[s], idx_v)

    @pl.when(c == 1)
    def _():
        pltpu.sync_copy(zeros_hbm.at[pl.ds(0, 320)],
                        dege_acc.at[pl.ds(s * 320, 320)])
        pltpu.sync_copy(he_hbm.at[s], idx_v)

    plsc.subcore_barrier()

    def scatter_all(acc):
        def chunk(j, carry):
            p = lax.rem(j, 4)

            @pl.when(j >= 4)
            def _():
                pltpu.make_async_copy(
                    ones_v, acc.at[idx_v.at[j]], ssem.at[p]).wait()

            pltpu.async_copy(ones_v, acc.at[idx_v.at[j]], ssem.at[p],
                             add=True)
            return carry
        lax.fori_loop(0, NCHD, chunk, 0)
        for k in range(4):
            pltpu.make_async_copy(
                ones_v, acc.at[idx_v.at[NCHD - 1]], ssem.at[k]).wait()

    @pl.when(c == 0)
    def _():
        scatter_all(degv_acc)

    @pl.when(c == 1)
    def _():
        scatter_all(dege_acc)

    plsc.subcore_barrier()

    @pl.when(c == 0)
    def _():
        pltpu.sync_copy(degv_acc.at[pl.ds(s * 632, 632)],
                        degv_out.at[pl.ds(s * 632, 632)])

    @pl.when(c == 1)
    def _():
        pltpu.sync_copy(dege_acc.at[pl.ds(s * 320, 320)],
                        dege_out.at[pl.ds(s * 320, 320)])


def _degrees(nd3, he3):
    return pl.kernel(
        _degree_kernel,
        out_type=[jax.ShapeDtypeStruct((NNP, DEGW), jnp.float32),
                  jax.ShapeDtypeStruct((NEP, DEGW), jnp.float32)],
        mesh=_mesh(),
        compiler_params=_sc_params,
        scratch_types=[
            pltpu.VMEM_SHARED((NNP, DEGW), jnp.float32),
            pltpu.VMEM_SHARED((NEP, DEGW), jnp.float32),
            pltpu.VMEM((NCHD, CHD), jnp.int32),
            pltpu.VMEM((CHD, DEGW), jnp.float32),
            pltpu.SemaphoreType.DMA((4,)),
        ],
    )(nd3, he3, jnp.ones((CHD, DEGW), jnp.float32),
      jnp.zeros((632, DEGW), jnp.float32))


def _mul_rows(dst, src, nrows):
    def row(i, carry):
        for k in range(DH // 16):
            sl = pl.ds(k * 16, 16)
            dst[i, sl] = dst[i, sl] * src[i, sl]
        return carry
    lax.fori_loop(0, nrows, row, 0)


def _make_spmm(n_seg, rows_per_tile, n_tbl, tbl_rows_per_tile, nbuf=NBUF,
               norm=False):
    def body(*refs):
        if norm:
            (t0_hbm, t1_hbm, dexp_hbm, gidx_hbm, sidx_hbm, o0, o1,
             acc, tbl, g_v, s_v, rowbuf, gsem, ssem) = refs
        else:
            (t0_hbm, t1_hbm, gidx_hbm, sidx_hbm, o0, o1,
             acc, tbl, g_v, s_v, rowbuf, gsem, ssem) = refs
        c = lax.axis_index("c")
        s = lax.axis_index("s")

        _zero_rows(rowbuf.at[0], CH, DH)
        base = s * rows_per_tile
        off = 0
        while off < rows_per_tile:
            n = min(CH, rows_per_tile - off)
            pltpu.sync_copy(rowbuf.at[0].at[pl.ds(0, n)],
                            acc.at[pl.ds(base + off, n)])
            off += n

        if not norm:
            tsl = pl.ds(s * tbl_rows_per_tile, tbl_rows_per_tile)

            @pl.when(c == 0)
            def _():
                pltpu.sync_copy(t0_hbm.at[tsl], tbl.at[tsl])

            @pl.when(c == 1)
            def _():
                pltpu.sync_copy(t1_hbm.at[tsl], tbl.at[tsl])
        else:
            off = 0
            while off < tbl_rows_per_tile:
                n = min(CH, tbl_rows_per_tile - off)
                rsl = pl.ds(s * tbl_rows_per_tile + off, n)
                bsl = pl.ds(0, n)

                @pl.when(c == 0)
                def _():
                    pltpu.sync_copy(t0_hbm.at[rsl], rowbuf.at[0].at[bsl])

                @pl.when(c == 1)
                def _():
                    pltpu.sync_copy(t1_hbm.at[rsl], rowbuf.at[0].at[bsl])

                pltpu.sync_copy(dexp_hbm.at[rsl], rowbuf.at[1].at[bsl])
                _mul_rows(rowbuf.at[0], rowbuf.at[1], n)
                pltpu.sync_copy(rowbuf.at[0].at[bsl], tbl.at[rsl])
                off += n

        pltpu.sync_copy(gidx_hbm.at[s], g_v)
        pltpu.sync_copy(sidx_hbm.at[s], s_v)
        plsc.subcore_barrier()

        def do_phase(t_ref):
            for g in range(nbuf - 1):
                pltpu.async_copy(t_ref.at[g_v.at[g]], rowbuf.at[g], gsem.at[g])

            def chunk(j, carry):
                p = lax.rem(j, nbuf)
                pq = lax.rem(j + nbuf - 1, nbuf)

                @pl.when(j >= 1)
                def _():
                    pltpu.make_async_copy(
                        rowbuf.at[pq], acc.at[s_v.at[j]], ssem.at[pq]).wait()

                @pl.when(j + nbuf - 1 < NCH16)
                def _():
                    pltpu.async_copy(t_ref.at[g_v.at[j + nbuf - 1]],
                                     rowbuf.at[pq], gsem.at[pq])

                pltpu.make_async_copy(
                    t_ref.at[g_v.at[j]], rowbuf.at[p], gsem.at[p]).wait()
                pltpu.async_copy(
                    rowbuf.at[p], acc.at[s_v.at[j]], ssem.at[p], add=True)
                return carry
            lax.fori_loop(0, NCH16, chunk, 0)
            lastp = (NCH16 - 1) % nbuf
            pltpu.make_async_copy(
                rowbuf.at[lastp], acc.at[s_v.at[NCH16 - 1]],
                ssem.at[lastp]).wait()

        do_phase(tbl)

        plsc.subcore_barrier()

        @pl.when(c == 0)
        def _():
            pltpu.sync_copy(acc.at[pl.ds(s * rows_per_tile, rows_per_tile)],
                            o0.at[pl.ds(s * rows_per_tile, rows_per_tile)])

        @pl.when(c == 1)
        def _():
            pltpu.sync_copy(acc.at[pl.ds(s * rows_per_tile, rows_per_tile)],
                            o1.at[pl.ds(s * rows_per_tile, rows_per_tile)])

    return pl.kernel(
        body,
        out_type=[jax.ShapeDtypeStruct((n_seg, DH), jnp.float32),
                  jax.ShapeDtypeStruct((n_seg, DH), jnp.float32)],
        mesh=_mesh(),
        compiler_params=_sc_params,
        scratch_types=[
            pltpu.VMEM_SHARED((n_seg, DH), jnp.float32),
            pltpu.VMEM_SHARED((n_tbl, DH), jnp.float32),
            pltpu.VMEM((NCH16, CH), jnp.int32),
            pltpu.VMEM((NCH16, CH), jnp.int32),
            pltpu.VMEM((nbuf, CH, DH), jnp.float32),
            pltpu.SemaphoreType.DMA((nbuf,)),
            pltpu.SemaphoreType.DMA((nbuf,)),
        ],
    )


def _xnorm_tc(x_ref, dv_ref, de_ref, o0, o1, dexp):
    d = dv_ref[pl.ds(0, NN), 0:1]
    inv = jnp.where(d > 0, 1.0 / d, 0.0)
    zpad = jnp.zeros((NNP - NN, DH), jnp.float32)
    o0[pl.ds(0, NN), :] = x_ref[:, :DH] * inv
    o0[pl.ds(NN, NNP - NN), :] = zpad
    o1[pl.ds(0, NN), :] = x_ref[:, DH:] * inv
    o1[pl.ds(NN, NNP - NN), :] = zpad
    de = de_ref[:, 0:1]
    inve = jnp.where(de > 0, 1.0 / de, 0.0)
    dexp[...] = jnp.broadcast_to(inve, (NEP, DH))


def kernel(X, Y, node_idx, hyperedge_idx):
    del Y
    pad = NIP - NI
    nd_flat = jnp.concatenate([node_idx, jnp.full((pad,), NN, jnp.int32)])
    he_flat = jnp.concatenate([hyperedge_idx, jnp.full((pad,), NE, jnp.int32)])
    nd = nd_flat.reshape(NS, NCH16, CH)
    he = he_flat.reshape(NS, NCH16, CH)
    degv, dege = _degrees(nd_flat.reshape(NS, NCHD, CHD),
                          he_flat.reshape(NS, NCHD, CHD))

    xn0, xn1, dexp = pl.pallas_call(
        _xnorm_tc,
        out_shape=[jax.ShapeDtypeStruct((NNP, DH), jnp.float32),
                   jax.ShapeDtypeStruct((NNP, DH), jnp.float32),
                   jax.ShapeDtypeStruct((NEP, DH), jnp.float32)],
    )(X, degv, dege)

    e0, e1 = _make_spmm(NEP, 320, NNP, 632, nbuf=7)(xn0, xn1, nd, he)

    n0, n1 = _make_spmm(NNP, 632, NEP, 320, nbuf=7, norm=True)(
        e0, e1, dexp, he, nd)

    node_feat = jnp.concatenate([n0[:NN], n1[:NN]], axis=1)
    edge_feat = jnp.concatenate([e0[:NE], e1[:NE]], axis=1)
    return node_feat, edge_feat


# speedup vs baseline: 1.1099x; 1.0067x over previous
import functools

import jax
import jax.numpy as jnp
from jax import lax
from jax.experimental import pallas as pl
from jax.experimental.pallas import tpu as pltpu
from jax.experimental.pallas import tpu_sc as plsc

NN = 10000
NE = 5000
NI = 320000
D = 128
DH = 64

NC = 2
NS = 16
CH = 64

NNP = 10112
NEP = 5120
NIP = 323584
NCH16 = NIP // (NS * CH)
CHD = 128
NCHD = NIP // (NS * CHD)
DEGW = 8
NBUF = 4

_mesh = functools.partial(
    plsc.VectorSubcoreMesh, core_axis_name="c", subcore_axis_name="s",
    num_cores=NC, num_subcores=NS)
_sc_params = pltpu.CompilerParams(use_tc_tiling_on_sc=False)


def _zero_rows(buf, nrows, width):
    def row(i, carry):
        for k in range(width // 16):
            buf[i, pl.ds(k * 16, 16)] = jnp.zeros((16,), jnp.float32)
        return carry
    lax.fori_loop(0, nrows, row, 0)


def _degree_kernel(nd_hbm, he_hbm, ones_hbm, zeros_hbm, degv_out, dege_out,
                   degv_acc, dege_acc, idx_v, ones_v, ssem):
    c = lax.axis_index("c")
    s = lax.axis_index("s")

    pltpu.sync_copy(ones_hbm, ones_v)

    @pl.when(c == 0)
    def _():
        pltpu.sync_copy(zeros_hbm, degv_acc.at[pl.ds(s * 632, 632)])
        pltpu.sync_copy(nd_hbm.at[s], idx_v)

    @pl.when(c == 1)
    def _():
        pltpu.sync_copy(zeros_hbm.at[pl.ds(0, 320)],
                        dege_acc.at[pl.ds(s * 320, 320)])
        pltpu.sync_copy(he_hbm.at[s], idx_v)

    plsc.subcore_barrier()

    def scatter_all(acc):
        def chunk(j, carry):
            p = lax.rem(j, 4)

            @pl.when(j >= 4)
            def _():
                pltpu.make_async_copy(
                    ones_v, acc.at[idx_v.at[j]], ssem.at[p]).wait()

            pltpu.async_copy(ones_v, acc.at[idx_v.at[j]], ssem.at[p],
                             add=True)
            return carry
        lax.fori_loop(0, NCHD, chunk, 0)
        for k in range(4):
            pltpu.make_async_copy(
                ones_v, acc.at[idx_v.at[NCHD - 1]], ssem.at[k]).wait()

    @pl.when(c == 0)
    def _():
        scatter_all(degv_acc)

    @pl.when(c == 1)
    def _():
        scatter_all(dege_acc)

    plsc.subcore_barrier()

    @pl.when(c == 0)
    def _():
        pltpu.sync_copy(degv_acc.at[pl.ds(s * 632, 632)],
                        degv_out.at[pl.ds(s * 632, 632)])

    @pl.when(c == 1)
    def _():
        pltpu.sync_copy(dege_acc.at[pl.ds(s * 320, 320)],
                        dege_out.at[pl.ds(s * 320, 320)])


def _degrees(nd3, he3):
    return pl.kernel(
        _degree_kernel,
        out_type=[jax.ShapeDtypeStruct((NNP, DEGW), jnp.float32),
                  jax.ShapeDtypeStruct((NEP, DEGW), jnp.float32)],
        mesh=_mesh(),
        compiler_params=_sc_params,
        scratch_types=[
            pltpu.VMEM_SHARED((NNP, DEGW), jnp.float32),
            pltpu.VMEM_SHARED((NEP, DEGW), jnp.float32),
            pltpu.VMEM((NCHD, CHD), jnp.int32),
            pltpu.VMEM((CHD, DEGW), jnp.float32),
            pltpu.SemaphoreType.DMA((4,)),
        ],
    )(nd3, he3, jnp.ones((CHD, DEGW), jnp.float32),
      jnp.zeros((632, DEGW), jnp.float32))


def _mul_rows(dst, src, nrows):
    def row(i, carry):
        for k in range(DH // 16):
            sl = pl.ds(k * 16, 16)
            dst[i, sl] = dst[i, sl] * src[i, sl]
        return carry
    lax.fori_loop(0, nrows, row, 0)


def _make_spmm(n_seg, rows_per_tile, n_tbl, tbl_rows_per_tile, nbuf=NBUF,
               norm=False):
    def body(*refs):
        if norm:
            (t0_hbm, t1_hbm, dexp_hbm, gidx_hbm, sidx_hbm, zeros_hbm, o0, o1,
             acc, tbl, g_v, s_v, rowbuf, gsem, ssem) = refs
        else:
            (t0_hbm, t1_hbm, gidx_hbm, sidx_hbm, zeros_hbm, o0, o1,
             acc, tbl, g_v, s_v, rowbuf, gsem, ssem) = refs
        c = lax.axis_index("c")
        s = lax.axis_index("s")

        startup = [
            pltpu.async_copy(zeros_hbm.at[pl.ds(0, rows_per_tile)],
                             acc.at[pl.ds(s * rows_per_tile, rows_per_tile)],
                             gsem.at[0]),
            pltpu.async_copy(gidx_hbm.at[s], g_v, gsem.at[1]),
            pltpu.async_copy(sidx_hbm.at[s], s_v, gsem.at[2]),
        ]

        if not norm:
            tsl = pl.ds(s * tbl_rows_per_tile, tbl_rows_per_tile)

            @pl.when(c == 0)
            def _():
                pltpu.async_copy(t0_hbm.at[tsl], tbl.at[tsl], ssem.at[0])

            @pl.when(c == 1)
            def _():
                pltpu.async_copy(t1_hbm.at[tsl], tbl.at[tsl], ssem.at[0])

            startup.append(pltpu.make_async_copy(
                t0_hbm.at[tsl], tbl.at[tsl], ssem.at[0]))
        else:
            off = 0
            while off < tbl_rows_per_tile:
                n = min(CH, tbl_rows_per_tile - off)
                rsl = pl.ds(s * tbl_rows_per_tile + off, n)
                bsl = pl.ds(0, n)

                @pl.when(c == 0)
                def _():
                    pltpu.sync_copy(t0_hbm.at[rsl], rowbuf.at[0].at[bsl])

                @pl.when(c == 1)
                def _():
                    pltpu.sync_copy(t1_hbm.at[rsl], rowbuf.at[0].at[bsl])

                pltpu.sync_copy(dexp_hbm.at[rsl], rowbuf.at[1].at[bsl])
                _mul_rows(rowbuf.at[0], rowbuf.at[1], n)
                pltpu.sync_copy(rowbuf.at[0].at[bsl], tbl.at[rsl])
                off += n

        for d in startup:
            d.wait()
        plsc.subcore_barrier()

        def do_phase(t_ref):
            for g in range(nbuf - 1):
                pltpu.async_copy(t_ref.at[g_v.at[g]], rowbuf.at[g], gsem.at[g])

            def chunk(j, carry):
                p = lax.rem(j, nbuf)
                pq = lax.rem(j + nbuf - 1, nbuf)

                @pl.when(j >= 1)
                def _():
                    pltpu.make_async_copy(
                        rowbuf.at[pq], acc.at[s_v.at[j]], ssem.at[pq]).wait()

                @pl.when(j + nbuf - 1 < NCH16)
                def _():
                    pltpu.async_copy(t_ref.at[g_v.at[j + nbuf - 1]],
                                     rowbuf.at[pq], gsem.at[pq])

                pltpu.make_async_copy(
                    t_ref.at[g_v.at[j]], rowbuf.at[p], gsem.at[p]).wait()
                pltpu.async_copy(
                    rowbuf.at[p], acc.at[s_v.at[j]], ssem.at[p], add=True)
                return carry
            lax.fori_loop(0, NCH16, chunk, 0)
            lastp = (NCH16 - 1) % nbuf
            pltpu.make_async_copy(
                rowbuf.at[lastp], acc.at[s_v.at[NCH16 - 1]],
                ssem.at[lastp]).wait()

        do_phase(tbl)

        plsc.subcore_barrier()

        @pl.when(c == 0)
        def _():
            pltpu.sync_copy(acc.at[pl.ds(s * rows_per_tile, rows_per_tile)],
                            o0.at[pl.ds(s * rows_per_tile, rows_per_tile)])

        @pl.when(c == 1)
        def _():
            pltpu.sync_copy(acc.at[pl.ds(s * rows_per_tile, rows_per_tile)],
                            o1.at[pl.ds(s * rows_per_tile, rows_per_tile)])

    return pl.kernel(
        body,
        out_type=[jax.ShapeDtypeStruct((n_seg, DH), jnp.float32),
                  jax.ShapeDtypeStruct((n_seg, DH), jnp.float32)],
        mesh=_mesh(),
        compiler_params=_sc_params,
        scratch_types=[
            pltpu.VMEM_SHARED((n_seg, DH), jnp.float32),
            pltpu.VMEM_SHARED((n_tbl, DH), jnp.float32),
            pltpu.VMEM((NCH16, CH), jnp.int32),
            pltpu.VMEM((NCH16, CH), jnp.int32),
            pltpu.VMEM((nbuf, CH, DH), jnp.float32),
            pltpu.SemaphoreType.DMA((nbuf,)),
            pltpu.SemaphoreType.DMA((nbuf,)),
        ],
    )


def _xnorm_tc(x_ref, dv_ref, de_ref, o0, o1, dexp):
    d = dv_ref[pl.ds(0, NN), 0:1]
    inv = jnp.where(d > 0, 1.0 / d, 0.0)
    zpad = jnp.zeros((NNP - NN, DH), jnp.float32)
    o0[pl.ds(0, NN), :] = x_ref[:, :DH] * inv
    o0[pl.ds(NN, NNP - NN), :] = zpad
    o1[pl.ds(0, NN), :] = x_ref[:, DH:] * inv
    o1[pl.ds(NN, NNP - NN), :] = zpad
    de = de_ref[:, 0:1]
    inve = jnp.where(de > 0, 1.0 / de, 0.0)
    dexp[...] = jnp.broadcast_to(inve, (NEP, DH))


def kernel(X, Y, node_idx, hyperedge_idx):
    del Y
    pad = NIP - NI
    nd_flat = jnp.concatenate([node_idx, jnp.full((pad,), NN, jnp.int32)])
    he_flat = jnp.concatenate([hyperedge_idx, jnp.full((pad,), NE, jnp.int32)])
    nd = nd_flat.reshape(NS, NCH16, CH)
    he = he_flat.reshape(NS, NCH16, CH)
    degv, dege = _degrees(nd_flat.reshape(NS, NCHD, CHD),
                          he_flat.reshape(NS, NCHD, CHD))

    xn0, xn1, dexp = pl.pallas_call(
        _xnorm_tc,
        out_shape=[jax.ShapeDtypeStruct((NNP, DH), jnp.float32),
                   jax.ShapeDtypeStruct((NNP, DH), jnp.float32),
                   jax.ShapeDtypeStruct((NEP, DH), jnp.float32)],
    )(X, degv, dege)

    e0, e1 = _make_spmm(NEP, 320, NNP, 632, nbuf=7)(xn0, xn1, nd, he, jnp.zeros((632, DH), jnp.float32))

    n0, n1 = _make_spmm(NNP, 632, NEP, 320, nbuf=7, norm=True)(
        e0, e1, dexp, he, nd, jnp.zeros((632, DH), jnp.float32))

    node_feat = jnp.concatenate([n0[:NN], n1[:NN]], axis=1)
    edge_feat = jnp.concatenate([e0[:NE], e1[:NE]], axis=1)
    return node_feat, edge_feat
